# Initial kernel scaffold; baseline (speedup 1.0000x reference)
import jax
import jax.numpy as jnp
from jax.experimental import pallas as pl


def kernel(hidden_states, W_router, Ws_gate, Ws_up, Ws_down, We_gate, We_up, We_down):
    raise NotImplementedError("write your pallas kernel here")


# speedup vs baseline: 1.5511x; 1.5511x over previous
import jax
import jax.numpy as jnp
from jax.experimental import pallas as pl
from jax.experimental.pallas import tpu as pltpu

HIDDEN = 1024
FFN = 2048
MOE_FFN = 512
E = 8
TOP_K = 2
T = 2048

TOK_BLK = 256


def _shared_router_body(x_ref, wr_ref, wg_ref, wu_ref, wd_ref, shared_ref, coef_ref):
    x = x_ref[...]
    xb = x.astype(jnp.bfloat16)

    logits = jnp.dot(x, wr_ref[...], preferred_element_type=jnp.float32)
    iota = jax.lax.broadcasted_iota(jnp.int32, logits.shape, 1)
    v1 = jnp.max(logits, axis=1, keepdims=True)
    idx1 = jnp.min(jnp.where(logits == v1, iota, E), axis=1, keepdims=True)
    sel1 = iota == idx1
    masked = jnp.where(sel1, -jnp.inf, logits)
    v2 = jnp.max(masked, axis=1, keepdims=True)
    idx2 = jnp.min(jnp.where(masked == v2, iota, E), axis=1, keepdims=True)
    sel2 = iota == idx2
    e2 = jnp.exp(v2 - v1)
    w1 = 1.0 / (1.0 + e2)
    w2 = 1.0 - w1
    coef_ref[...] = jnp.where(sel1, w1, 0.0) + jnp.where(sel2, w2, 0.0)

    g = jnp.dot(xb, wg_ref[...], preferred_element_type=jnp.float32)
    u = jnp.dot(xb, wu_ref[...], preferred_element_type=jnp.float32)
    h = (g * jax.nn.sigmoid(g) * u).astype(jnp.bfloat16)
    shared_ref[...] = jnp.dot(h, wd_ref[...], preferred_element_type=jnp.float32)


def _experts_body(x_ref, coef_ref, shared_ref, wg_ref, wu_ref, wd_ref, out_ref):
    xb = x_ref[...].astype(jnp.bfloat16)
    coef = coef_ref[...]
    acc = shared_ref[...]
    for e in range(E):
        g = jnp.dot(xb, wg_ref[e], preferred_element_type=jnp.float32)
        u = jnp.dot(xb, wu_ref[e], preferred_element_type=jnp.float32)
        h = (g * jax.nn.sigmoid(g) * u).astype(jnp.bfloat16)
        y = jnp.dot(h, wd_ref[e], preferred_element_type=jnp.float32)
        acc = acc + coef[:, e : e + 1] * y
    out_ref[...] = acc


def kernel(hidden_states, W_router, Ws_gate, Ws_up, Ws_down, We_gate, We_up, We_down):
    orig_shape = hidden_states.shape
    x = hidden_states.reshape(-1, HIDDEN)
    n_tok_blocks = T // TOK_BLK

    shared, coef = pl.pallas_call(
        _shared_router_body,
        grid=(n_tok_blocks,),
        in_specs=[
            pl.BlockSpec((TOK_BLK, HIDDEN), lambda t: (t, 0)),
            pl.BlockSpec((HIDDEN, E), lambda t: (0, 0)),
            pl.BlockSpec((HIDDEN, FFN), lambda t: (0, 0)),
            pl.BlockSpec((HIDDEN, FFN), lambda t: (0, 0)),
            pl.BlockSpec((FFN, HIDDEN), lambda t: (0, 0)),
        ],
        out_specs=[
            pl.BlockSpec((TOK_BLK, HIDDEN), lambda t: (t, 0)),
            pl.BlockSpec((TOK_BLK, E), lambda t: (t, 0)),
        ],
        out_shape=[
            jax.ShapeDtypeStruct((T, HIDDEN), jnp.float32),
            jax.ShapeDtypeStruct((T, E), jnp.float32),
        ],
    )(
        x,
        W_router,
        Ws_gate.astype(jnp.bfloat16),
        Ws_up.astype(jnp.bfloat16),
        Ws_down.astype(jnp.bfloat16),
    )

    out = pl.pallas_call(
        _experts_body,
        grid=(n_tok_blocks,),
        in_specs=[
            pl.BlockSpec((TOK_BLK, HIDDEN), lambda t: (t, 0)),
            pl.BlockSpec((TOK_BLK, E), lambda t: (t, 0)),
            pl.BlockSpec((TOK_BLK, HIDDEN), lambda t: (t, 0)),
            pl.BlockSpec((E, HIDDEN, MOE_FFN), lambda t: (0, 0, 0)),
            pl.BlockSpec((E, HIDDEN, MOE_FFN), lambda t: (0, 0, 0)),
            pl.BlockSpec((E, MOE_FFN, HIDDEN), lambda t: (0, 0, 0)),
        ],
        out_specs=pl.BlockSpec((TOK_BLK, HIDDEN), lambda t: (t, 0)),
        out_shape=jax.ShapeDtypeStruct((T, HIDDEN), jnp.float32),
    )(
        x,
        coef,
        shared,
        We_gate.astype(jnp.bfloat16),
        We_up.astype(jnp.bfloat16),
        We_down.astype(jnp.bfloat16),
    )
    return out.reshape(orig_shape)
